# baseline (speedup 1.0000x reference)
import functools
import math

import jax
import jax.numpy as jnp
from jax import lax
from jax.experimental import pallas as pl
from jax.experimental.pallas import tpu as pltpu
from jax.experimental.pallas import tpu_sc as plsc

DIM = 768
E = 8
K = 2
H = 4 * DIM
B, N = 8, 196
T = B * N
CAP = int(math.ceil(1.25 * T * K / E))
CP = 512
DUMP = CP - 1
NSUB = 28
TW = T // NSUB
RB = 392
HT = 3072
EH = 4


def _gating_body(x_ref, wg_ref, bg_ref, pr_ref, i0_ref, i1_ref, w0_ref,
                 w1_ref, me_ref, cnt_ref):
    logits = jnp.dot(x_ref[...], wg_ref[...], preferred_element_type=jnp.float32)
    logits = logits + bg_ref[...]
    gmax = jnp.max(logits, axis=1, keepdims=True)
    z = jnp.exp(logits - gmax)
    gates = z / jnp.sum(z, axis=1, keepdims=True)
    e_iota = lax.broadcasted_iota(jnp.int32, (T, E), 1)
    v0 = jnp.max(gates, axis=1, keepdims=True)
    i0 = jnp.min(jnp.where(gates == v0, e_iota, E), axis=1, keepdims=True)
    g1 = jnp.where(e_iota == i0, -1.0, gates)
    v1 = jnp.max(g1, axis=1, keepdims=True)
    i1 = jnp.min(jnp.where(g1 == v1, e_iota, E), axis=1, keepdims=True)
    ws = v0 + v1
    me = jnp.logical_or(e_iota == i0, e_iota == i1).astype(jnp.float32)
    pr_ref[...] = v0
    i0_ref[...] = i0
    i1_ref[...] = i1
    w0_ref[...] = v0 / ws
    w1_ref[...] = v1 / ws
    me_ref[...] = me
    cnt_ref[...] = jnp.minimum(jnp.sum(me, axis=0, keepdims=True), float(CAP))


_gating_call = pl.pallas_call(
    _gating_body,
    out_shape=[
        jax.ShapeDtypeStruct((T, 1), jnp.float32),
        jax.ShapeDtypeStruct((T, 1), jnp.int32),
        jax.ShapeDtypeStruct((T, 1), jnp.int32),
        jax.ShapeDtypeStruct((T, 1), jnp.float32),
        jax.ShapeDtypeStruct((T, 1), jnp.float32),
        jax.ShapeDtypeStruct((T, E), jnp.float32),
        jax.ShapeDtypeStruct((1, E), jnp.float32),
    ],
)


def _cap_body(prc_ref, prr_ref, me_ref, i0_ref, i1_ref, w0_ref, w1_ref,
              dA0_ref, dA1_ref, dB0_ref, dB1_ref, c0_ref, c1_ref,
              wbA0_ref, wbA1_ref, wbB0_ref, wbB1_ref):
    b = pl.program_id(0)
    prc = prc_ref[...]
    prr = prr_ref[...]
    gt = (prr > prc).astype(jnp.float32)
    tcol = lax.broadcasted_iota(jnp.int32, (RB, T), 1)
    trow = lax.broadcasted_iota(jnp.int32, (RB, T), 0) + b * RB
    eqlt = jnp.logical_and(prr == prc, tcol < trow).astype(jnp.float32)
    cnt = jnp.dot(gt + eqlt, me_ref[...], preferred_element_type=jnp.float32)
    e_iota = lax.broadcasted_iota(jnp.int32, (RB, E), 1)
    i0 = i0_ref[...]
    i1 = i1_ref[...]
    pos0 = jnp.sum(jnp.where(e_iota == i0, cnt, 0.0), axis=1,
                   keepdims=True).astype(jnp.int32)
    pos1 = jnp.sum(jnp.where(e_iota == i1, cnt, 0.0), axis=1,
                   keepdims=True).astype(jnp.int32)
    s0 = jnp.minimum(pos0, CAP - 1)
    s1 = jnp.minimum(pos1, CAP - 1)
    k0 = pos0 < CAP
    k1 = pos1 < CAP
    inA0 = jnp.logical_and(k0, i0 < EH)
    inA1 = jnp.logical_and(k1, i1 < EH)
    inB0 = jnp.logical_and(k0, i0 >= EH)
    inB1 = jnp.logical_and(k1, i1 >= EH)
    dA0_ref[...] = jnp.where(inA0, i0 * CP + s0, DUMP)
    dA1_ref[...] = jnp.where(inA1, i1 * CP + s1, DUMP)
    dB0_ref[...] = jnp.where(inB0, (i0 - EH) * CP + s0, DUMP)
    dB1_ref[...] = jnp.where(inB1, (i1 - EH) * CP + s1, DUMP)
    c0_ref[...] = jnp.where(k0, i0 * CP + s0, DUMP)
    c1_ref[...] = jnp.where(k1, i1 * CP + s1, DUMP)
    wbA0_ref[...] = jnp.broadcast_to(jnp.where(inA0, w0_ref[...], 0.0), (RB, 128))
    wbA1_ref[...] = jnp.broadcast_to(jnp.where(inA1, w1_ref[...], 0.0), (RB, 128))
    wbB0_ref[...] = jnp.broadcast_to(jnp.where(inB0, w0_ref[...], 0.0), (RB, 128))
    wbB1_ref[...] = jnp.broadcast_to(jnp.where(inB1, w1_ref[...], 0.0), (RB, 128))


_col_spec = pl.BlockSpec((RB, 1), lambda b: (b, 0))
_cap_call = pl.pallas_call(
    _cap_body,
    grid=(T // RB,),
    in_specs=[
        _col_spec,
        pl.BlockSpec((1, T), lambda b: (0, 0)),
        pl.BlockSpec((T, E), lambda b: (0, 0)),
        _col_spec, _col_spec, _col_spec, _col_spec,
    ],
    out_specs=[_col_spec] * 6 + [pl.BlockSpec((RB, 128), lambda b: (b, 0))] * 4,
    out_shape=[
        jax.ShapeDtypeStruct((T, 1), jnp.int32),
        jax.ShapeDtypeStruct((T, 1), jnp.int32),
        jax.ShapeDtypeStruct((T, 1), jnp.int32),
        jax.ShapeDtypeStruct((T, 1), jnp.int32),
        jax.ShapeDtypeStruct((T, 1), jnp.int32),
        jax.ShapeDtypeStruct((T, 1), jnp.int32),
        jax.ShapeDtypeStruct((T, 128), jnp.float32),
        jax.ShapeDtypeStruct((T, 128), jnp.float32),
        jax.ShapeDtypeStruct((T, 128), jnp.float32),
        jax.ShapeDtypeStruct((T, 128), jnp.float32),
    ],
)


_MESH = plsc.VectorSubcoreMesh(core_axis_name="c", subcore_axis_name="s")
_INFO = plsc.get_sparse_core_info()
_NC = _INFO.num_cores


@functools.partial(
    pl.kernel,
    mesh=_MESH,
    out_type=(
        jax.ShapeDtypeStruct((EH * CP, DIM), jnp.float32),
        jax.ShapeDtypeStruct((EH * CP, 128), jnp.float32),
    ),
    scratch_types=[
        pltpu.VMEM((TW, DIM), jnp.float32),
        pltpu.VMEM((TW,), jnp.int32),
        pltpu.VMEM((TW,), jnp.int32),
        pltpu.VMEM((TW, 128), jnp.float32),
        pltpu.VMEM((TW, 128), jnp.float32),
        pltpu.SemaphoreType.DMA,
    ],
)
def _scatter_call(x_hbm, d0_hbm, d1_hbm, wb0_hbm, wb1_hbm, ei_hbm, sw_hbm,
                  x_v, d0_v, d1_v, w0_v, w1_v, sem):
    wid = lax.axis_index("s") * _NC + lax.axis_index("c")

    @pl.when(wid < NSUB)
    def _():
        base = wid * TW
        pltpu.sync_copy(x_hbm.at[pl.ds(base, TW)], x_v)
        pltpu.sync_copy(d0_hbm.at[pl.ds(base, TW)], d0_v)
        pltpu.sync_copy(d1_hbm.at[pl.ds(base, TW)], d1_v)
        pltpu.sync_copy(wb0_hbm.at[pl.ds(base, TW)], w0_v)
        pltpu.sync_copy(wb1_hbm.at[pl.ds(base, TW)], w1_v)
        cp0 = pltpu.async_copy(x_v, ei_hbm.at[d0_v], sem)
        cp1 = pltpu.async_copy(x_v, ei_hbm.at[d1_v], sem)
        cp2 = pltpu.async_copy(w0_v, sw_hbm.at[d0_v], sem)
        cp3 = pltpu.async_copy(w1_v, sw_hbm.at[d1_v], sem)
        cp0.wait()
        cp1.wait()
        cp2.wait()
        cp3.wait()


_SQRT1_2 = 0.7071067811865476
_HN = H // HT


def _ffn_body(cnt_ref, x_ref, w1_ref, b1_ref, w2_ref, b2_ref, sw_ref, *rest):
    o_ref = rest[-1]
    cnt = cnt_ref[0, 0, 0].astype(jnp.int32)
    rows = lax.broadcasted_iota(jnp.int32, (CP, 1), 0)
    xb = jnp.where(rows < cnt, x_ref[0], 0.0).astype(jnp.bfloat16)
    hb = jnp.dot(xb, w1_ref[0].astype(jnp.bfloat16),
                 preferred_element_type=jnp.float32) + b1_ref[0]
    hb = 0.5 * hb * (1.0 + lax.erf(hb * _SQRT1_2))
    yb = jnp.dot(hb.astype(jnp.bfloat16), w2_ref[0].astype(jnp.bfloat16),
                 preferred_element_type=jnp.float32)
    o_ref[0] = (yb + b2_ref[0]) * sw_ref[0][:, 0:1]


def _make_ffn(off, alias):
    in_specs = [
        pl.BlockSpec((1, 1, 1), lambda e, h: (e + off, 0, 0)),
        pl.BlockSpec((1, CP, DIM), lambda e, h: (e, 0, 0)),
        pl.BlockSpec((1, DIM, HT), lambda e, h: (e + off, 0, h)),
        pl.BlockSpec((1, 1, HT), lambda e, h: (e + off, 0, h)),
        pl.BlockSpec((1, HT, DIM), lambda e, h: (e + off, h, 0)),
        pl.BlockSpec((1, 1, DIM), lambda e, h: (e + off, 0, 0)),
        pl.BlockSpec((1, CP, 128), lambda e, h: (e, 0, 0)),
    ]
    if alias:
        in_specs.append(pl.BlockSpec(memory_space=pl.ANY))
    return pl.pallas_call(
        _ffn_body,
        grid=(EH, _HN),
        in_specs=in_specs,
        out_specs=pl.BlockSpec((1, CP, DIM), lambda e, h: (e + off, 0, 0)),
        out_shape=jax.ShapeDtypeStruct((E, CP, DIM), jnp.float32),
        input_output_aliases={7: 0} if alias else {},
        compiler_params=pltpu.CompilerParams(
            dimension_semantics=("parallel", "arbitrary")),
    )


_ffn_a = _make_ffn(0, alias=False)
_ffn_b = _make_ffn(EH, alias=True)


@functools.partial(
    pl.kernel,
    mesh=_MESH,
    out_type=jax.ShapeDtypeStruct((T, DIM), jnp.float32),
    scratch_types=[
        pltpu.VMEM((TW, DIM), jnp.float32),
        pltpu.VMEM((TW, DIM), jnp.float32),
        pltpu.VMEM((TW,), jnp.int32),
        pltpu.VMEM((TW,), jnp.int32),
        pltpu.SemaphoreType.DMA,
    ],
)
def _combine_call(y_hbm, c0_hbm, c1_hbm, out_hbm, r0, r1, i0_v, i1_v, sem):
    wid = lax.axis_index("s") * _NC + lax.axis_index("c")

    @pl.when(wid < NSUB)
    def _():
        base = wid * TW
        pltpu.sync_copy(c0_hbm.at[pl.ds(base, TW)], i0_v)
        pltpu.sync_copy(c1_hbm.at[pl.ds(base, TW)], i1_v)
        cp0 = pltpu.async_copy(y_hbm.at[i0_v], r0, sem)
        cp1 = pltpu.async_copy(y_hbm.at[i1_v], r1, sem)
        cp0.wait()
        cp1.wait()

        def body(i, carry):
            for c in range(DIM // 16):
                sl = pl.ds(c * 16, 16)
                r0[i, sl] = r0[i, sl] + r1[i, sl]
            return carry

        lax.fori_loop(0, TW, body, 0)
        pltpu.sync_copy(r0, out_hbm.at[pl.ds(base, TW)])


def kernel(x, Wg, bg, W1, b1, W2, b2):
    x_flat = x.reshape(T, DIM)
    pr, i0, i1, w0, w1, me, counts = _gating_call(x_flat, Wg, bg.reshape(1, E))
    (dA0, dA1, dB0, dB1, c0, c1,
     wbA0, wbA1, wbB0, wbB1) = _cap_call(pr, pr.reshape(1, T), me, i0, i1,
                                         w0, w1)
    eiA, swA = _scatter_call(x_flat, dA0.reshape(T), dA1.reshape(T),
                             wbA0, wbA1)
    eiB, swB = _scatter_call(x_flat, dB0.reshape(T), dB1.reshape(T),
                             wbB0, wbB1)
    cr = counts.reshape(E, 1, 1)
    b1r = b1.reshape(E, 1, H)
    b2r = b2.reshape(E, 1, DIM)
    y0 = _ffn_a(cr, eiA.reshape(EH, CP, DIM), W1, b1r, W2, b2r,
                swA.reshape(EH, CP, 128))
    y = _ffn_b(cr, eiB.reshape(EH, CP, DIM), W1, b1r, W2, b2r,
               swB.reshape(EH, CP, 128), y0)
    out = _combine_call(y.reshape(E * CP, DIM), c0.reshape(T), c1.reshape(T))
    return out.reshape(B, N, DIM)


# speedup vs baseline: 2.3056x; 2.3056x over previous
import functools
import math

import jax
import jax.numpy as jnp
from jax import lax
from jax.experimental import pallas as pl
from jax.experimental.pallas import tpu as pltpu
from jax.experimental.pallas import tpu_sc as plsc

DIM = 768
E = 8
K = 2
H = 4 * DIM
B, N = 8, 196
T = B * N
CAP = int(math.ceil(1.25 * T * K / E))
CP = 512
DUMP = CP - 1
NSUB = 28
TW = T // NSUB
RB = 392
HT = 3072
EH = 4


def _gating_body(x_ref, wg_ref, bg_ref, pr_ref, i0_ref, i1_ref, w0_ref,
                 w1_ref, me_ref, cnt_ref):
    logits = jnp.dot(x_ref[...], wg_ref[...], preferred_element_type=jnp.float32)
    logits = logits + bg_ref[...]
    gmax = jnp.max(logits, axis=1, keepdims=True)
    z = jnp.exp(logits - gmax)
    gates = z / jnp.sum(z, axis=1, keepdims=True)
    e_iota = lax.broadcasted_iota(jnp.int32, (T, E), 1)
    v0 = jnp.max(gates, axis=1, keepdims=True)
    i0 = jnp.min(jnp.where(gates == v0, e_iota, E), axis=1, keepdims=True)
    g1 = jnp.where(e_iota == i0, -1.0, gates)
    v1 = jnp.max(g1, axis=1, keepdims=True)
    i1 = jnp.min(jnp.where(g1 == v1, e_iota, E), axis=1, keepdims=True)
    ws = v0 + v1
    me = jnp.logical_or(e_iota == i0, e_iota == i1).astype(jnp.float32)
    pr_ref[...] = v0
    i0_ref[...] = i0
    i1_ref[...] = i1
    w0_ref[...] = v0 / ws
    w1_ref[...] = v1 / ws
    me_ref[...] = me
    cnt_ref[...] = jnp.minimum(jnp.sum(me, axis=0, keepdims=True), float(CAP))


_gating_call = pl.pallas_call(
    _gating_body,
    out_shape=[
        jax.ShapeDtypeStruct((T, 1), jnp.float32),
        jax.ShapeDtypeStruct((T, 1), jnp.int32),
        jax.ShapeDtypeStruct((T, 1), jnp.int32),
        jax.ShapeDtypeStruct((T, 1), jnp.float32),
        jax.ShapeDtypeStruct((T, 1), jnp.float32),
        jax.ShapeDtypeStruct((T, E), jnp.float32),
        jax.ShapeDtypeStruct((1, E), jnp.float32),
    ],
)


def _cap_body(prc_ref, prr_ref, me_ref, i0_ref, i1_ref, w0_ref, w1_ref,
              dA0_ref, dA1_ref, dB0_ref, dB1_ref, c0_ref, c1_ref,
              wbA0_ref, wbA1_ref, wbB0_ref, wbB1_ref):
    b = pl.program_id(0)
    prc = prc_ref[...]
    prr = prr_ref[...]
    gt = (prr > prc).astype(jnp.float32)
    tcol = lax.broadcasted_iota(jnp.int32, (RB, T), 1)
    trow = lax.broadcasted_iota(jnp.int32, (RB, T), 0) + b * RB
    eqlt = jnp.logical_and(prr == prc, tcol < trow).astype(jnp.float32)
    cnt = jnp.dot(gt + eqlt, me_ref[...], preferred_element_type=jnp.float32)
    e_iota = lax.broadcasted_iota(jnp.int32, (RB, E), 1)
    i0 = i0_ref[...]
    i1 = i1_ref[...]
    pos0 = jnp.sum(jnp.where(e_iota == i0, cnt, 0.0), axis=1,
                   keepdims=True).astype(jnp.int32)
    pos1 = jnp.sum(jnp.where(e_iota == i1, cnt, 0.0), axis=1,
                   keepdims=True).astype(jnp.int32)
    s0 = jnp.minimum(pos0, CAP - 1)
    s1 = jnp.minimum(pos1, CAP - 1)
    k0 = pos0 < CAP
    k1 = pos1 < CAP
    inA0 = jnp.logical_and(k0, i0 < EH)
    inA1 = jnp.logical_and(k1, i1 < EH)
    inB0 = jnp.logical_and(k0, i0 >= EH)
    inB1 = jnp.logical_and(k1, i1 >= EH)
    tidc = b * RB + lax.broadcasted_iota(jnp.int32, (RB, 1), 0)
    dump = EH * CP + tidc // TW
    dA0_ref[...] = jnp.where(inA0, i0 * CP + s0, dump)
    dA1_ref[...] = jnp.where(inA1, i1 * CP + s1, dump)
    dB0_ref[...] = jnp.where(inB0, (i0 - EH) * CP + s0, dump)
    dB1_ref[...] = jnp.where(inB1, (i1 - EH) * CP + s1, dump)
    c0_ref[...] = jnp.where(k0, i0 * CP + s0, DUMP)
    c1_ref[...] = jnp.where(k1, i1 * CP + s1, DUMP)
    wbA0_ref[...] = jnp.broadcast_to(jnp.where(inA0, w0_ref[...], 0.0), (RB, 128))
    wbA1_ref[...] = jnp.broadcast_to(jnp.where(inA1, w1_ref[...], 0.0), (RB, 128))
    wbB0_ref[...] = jnp.broadcast_to(jnp.where(inB0, w0_ref[...], 0.0), (RB, 128))
    wbB1_ref[...] = jnp.broadcast_to(jnp.where(inB1, w1_ref[...], 0.0), (RB, 128))


_col_spec = pl.BlockSpec((RB, 1), lambda b: (b, 0))
_cap_call = pl.pallas_call(
    _cap_body,
    grid=(T // RB,),
    in_specs=[
        _col_spec,
        pl.BlockSpec((1, T), lambda b: (0, 0)),
        pl.BlockSpec((T, E), lambda b: (0, 0)),
        _col_spec, _col_spec, _col_spec, _col_spec,
    ],
    out_specs=[_col_spec] * 6 + [pl.BlockSpec((RB, 128), lambda b: (b, 0))] * 4,
    out_shape=[
        jax.ShapeDtypeStruct((T, 1), jnp.int32),
        jax.ShapeDtypeStruct((T, 1), jnp.int32),
        jax.ShapeDtypeStruct((T, 1), jnp.int32),
        jax.ShapeDtypeStruct((T, 1), jnp.int32),
        jax.ShapeDtypeStruct((T, 1), jnp.int32),
        jax.ShapeDtypeStruct((T, 1), jnp.int32),
        jax.ShapeDtypeStruct((T, 128), jnp.float32),
        jax.ShapeDtypeStruct((T, 128), jnp.float32),
        jax.ShapeDtypeStruct((T, 128), jnp.float32),
        jax.ShapeDtypeStruct((T, 128), jnp.float32),
    ],
)


_MESH = plsc.VectorSubcoreMesh(core_axis_name="c", subcore_axis_name="s")
_INFO = plsc.get_sparse_core_info()
_NC = _INFO.num_cores


@functools.partial(
    pl.kernel,
    mesh=_MESH,
    out_type=(
        jax.ShapeDtypeStruct((EH * CP + NSUB + 4, DIM), jnp.float32),
        jax.ShapeDtypeStruct((EH * CP + NSUB + 4, 128), jnp.float32),
    ),
    scratch_types=[
        pltpu.VMEM((TW, DIM), jnp.float32),
        pltpu.VMEM((TW,), jnp.int32),
        pltpu.VMEM((TW,), jnp.int32),
        pltpu.VMEM((TW, 128), jnp.float32),
        pltpu.VMEM((TW, 128), jnp.float32),
        pltpu.SemaphoreType.DMA,
    ],
)
def _scatter_call(x_hbm, d0_hbm, d1_hbm, wb0_hbm, wb1_hbm, ei_hbm, sw_hbm,
                  x_v, d0_v, d1_v, w0_v, w1_v, sem):
    wid = lax.axis_index("s") * _NC + lax.axis_index("c")

    @pl.when(wid < NSUB)
    def _():
        base = wid * TW
        pltpu.sync_copy(x_hbm.at[pl.ds(base, TW)], x_v)
        pltpu.sync_copy(d0_hbm.at[pl.ds(base, TW)], d0_v)
        pltpu.sync_copy(d1_hbm.at[pl.ds(base, TW)], d1_v)
        pltpu.sync_copy(wb0_hbm.at[pl.ds(base, TW)], w0_v)
        pltpu.sync_copy(wb1_hbm.at[pl.ds(base, TW)], w1_v)
        cp0 = pltpu.async_copy(x_v, ei_hbm.at[d0_v], sem)
        cp1 = pltpu.async_copy(x_v, ei_hbm.at[d1_v], sem)
        cp2 = pltpu.async_copy(w0_v, sw_hbm.at[d0_v], sem)
        cp3 = pltpu.async_copy(w1_v, sw_hbm.at[d1_v], sem)
        cp0.wait()
        cp1.wait()
        cp2.wait()
        cp3.wait()


_SQRT1_2 = 0.7071067811865476
_HN = H // HT


def _ffn_body(cnt_ref, x_ref, w1_ref, b1_ref, w2_ref, b2_ref, sw_ref, *rest):
    o_ref = rest[-1]
    cnt = cnt_ref[0, 0, 0].astype(jnp.int32)
    rows = lax.broadcasted_iota(jnp.int32, (CP, 1), 0)
    live = rows < cnt
    xb = jnp.where(live, x_ref[...], 0.0).astype(jnp.bfloat16)
    hb = jnp.dot(xb, w1_ref[0].astype(jnp.bfloat16),
                 preferred_element_type=jnp.float32) + b1_ref[0]
    hb = 0.5 * hb * (1.0 + lax.erf(hb * _SQRT1_2))
    yb = jnp.dot(hb.astype(jnp.bfloat16), w2_ref[0].astype(jnp.bfloat16),
                 preferred_element_type=jnp.float32)
    swc = jnp.where(live, sw_ref[...][:, 0:1], 0.0)
    o_ref[0] = (yb + b2_ref[0]) * swc


def _make_ffn(off, alias):
    in_specs = [
        pl.BlockSpec((1, 1, 1), lambda e, h: (e + off, 0, 0)),
        pl.BlockSpec((CP, DIM), lambda e, h: (e, 0)),
        pl.BlockSpec((1, DIM, HT), lambda e, h: (e + off, 0, h)),
        pl.BlockSpec((1, 1, HT), lambda e, h: (e + off, 0, h)),
        pl.BlockSpec((1, HT, DIM), lambda e, h: (e + off, h, 0)),
        pl.BlockSpec((1, 1, DIM), lambda e, h: (e + off, 0, 0)),
        pl.BlockSpec((CP, 128), lambda e, h: (e, 0)),
    ]
    if alias:
        in_specs.append(pl.BlockSpec(memory_space=pl.ANY))
    return pl.pallas_call(
        _ffn_body,
        grid=(EH, _HN),
        in_specs=in_specs,
        out_specs=pl.BlockSpec((1, CP, DIM), lambda e, h: (e + off, 0, 0)),
        out_shape=jax.ShapeDtypeStruct((E, CP, DIM), jnp.float32),
        input_output_aliases={7: 0} if alias else {},
        compiler_params=pltpu.CompilerParams(
            dimension_semantics=("parallel", "arbitrary")),
    )


_ffn_a = _make_ffn(0, alias=False)
_ffn_b = _make_ffn(EH, alias=True)


@functools.partial(
    pl.kernel,
    mesh=_MESH,
    out_type=jax.ShapeDtypeStruct((T, DIM), jnp.float32),
    scratch_types=[
        pltpu.VMEM((TW, DIM), jnp.float32),
        pltpu.VMEM((TW, DIM), jnp.float32),
        pltpu.VMEM((TW,), jnp.int32),
        pltpu.VMEM((TW,), jnp.int32),
        pltpu.SemaphoreType.DMA,
    ],
)
def _combine_call(y_hbm, c0_hbm, c1_hbm, out_hbm, r0, r1, i0_v, i1_v, sem):
    wid = lax.axis_index("s") * _NC + lax.axis_index("c")

    @pl.when(wid < NSUB)
    def _():
        base = wid * TW
        pltpu.sync_copy(c0_hbm.at[pl.ds(base, TW)], i0_v)
        pltpu.sync_copy(c1_hbm.at[pl.ds(base, TW)], i1_v)
        cp0 = pltpu.async_copy(y_hbm.at[i0_v], r0, sem)
        cp1 = pltpu.async_copy(y_hbm.at[i1_v], r1, sem)
        cp0.wait()
        cp1.wait()

        def body(i, carry):
            for c in range(DIM // 16):
                sl = pl.ds(c * 16, 16)
                r0[i, sl] = r0[i, sl] + r1[i, sl]
            return carry

        lax.fori_loop(0, TW, body, 0)
        pltpu.sync_copy(r0, out_hbm.at[pl.ds(base, TW)])


def kernel(x, Wg, bg, W1, b1, W2, b2):
    x_flat = x.reshape(T, DIM)
    pr, i0, i1, w0, w1, me, counts = _gating_call(x_flat, Wg, bg.reshape(1, E))
    (dA0, dA1, dB0, dB1, c0, c1,
     wbA0, wbA1, wbB0, wbB1) = _cap_call(pr, pr.reshape(1, T), me, i0, i1,
                                         w0, w1)
    eiA, swA = _scatter_call(x_flat, dA0.reshape(T), dA1.reshape(T),
                             wbA0, wbA1)
    eiB, swB = _scatter_call(x_flat, dB0.reshape(T), dB1.reshape(T),
                             wbB0, wbB1)
    cr = counts.reshape(E, 1, 1)
    b1r = b1.reshape(E, 1, H)
    b2r = b2.reshape(E, 1, DIM)
    y0 = _ffn_a(cr, eiA, W1, b1r, W2, b2r, swA)
    y = _ffn_b(cr, eiB, W1, b1r, W2, b2r, swB, y0)
    out = _combine_call(y.reshape(E * CP, DIM), c0.reshape(T), c1.reshape(T))
    return out.reshape(B, N, DIM)


# speedup vs baseline: 2.7576x; 1.1961x over previous
import functools
import math

import jax
import jax.numpy as jnp
from jax import lax
from jax.experimental import pallas as pl
from jax.experimental.pallas import tpu as pltpu
from jax.experimental.pallas import tpu_sc as plsc

DIM = 768
E = 8
K = 2
H = 4 * DIM
B, N = 8, 196
T = B * N
CAP = int(math.ceil(1.25 * T * K / E))
CP = 512
DUMP = CP - 1
NSUB = 28
TW = T // NSUB
RB = 392
HT = 3072


def _route_body(x_ref, wg_ref, bg_ref, d0_ref, d1_ref, wb0_ref, wb1_ref,
                cnt_ref):
    logits = jnp.dot(x_ref[...], wg_ref[...], preferred_element_type=jnp.float32)
    logits = logits + bg_ref[...]
    gmax = jnp.max(logits, axis=1, keepdims=True)
    z = jnp.exp(logits - gmax)
    gates = z / jnp.sum(z, axis=1, keepdims=True)
    e_iota = lax.broadcasted_iota(jnp.int32, (T, E), 1)
    v0 = jnp.max(gates, axis=1, keepdims=True)
    i0 = jnp.min(jnp.where(gates == v0, e_iota, E), axis=1, keepdims=True)
    g1 = jnp.where(e_iota == i0, -1.0, gates)
    v1 = jnp.max(g1, axis=1, keepdims=True)
    i1 = jnp.min(jnp.where(g1 == v1, e_iota, E), axis=1, keepdims=True)
    ws = v0 + v1
    w0 = v0 / ws
    w1 = v1 / ws
    me = jnp.logical_or(e_iota == i0, e_iota == i1).astype(jnp.float32)
    cnt_ref[...] = jnp.minimum(jnp.sum(me, axis=0, keepdims=True), float(CAP))
    pr = v0
    prr_parts = []
    for b in range(T // RB):
        crow = lax.broadcasted_iota(jnp.int32, (T, RB), 0)
        ccol = lax.broadcasted_iota(jnp.int32, (T, RB), 1) + b * RB
        ident = (crow == ccol).astype(jnp.float32)
        prr_parts.append(lax.dot_general(
            pr, ident, (((0,), (0,)), ((), ())),
            precision=lax.Precision.HIGHEST,
            preferred_element_type=jnp.float32))
    prr = jnp.concatenate(prr_parts, axis=1)
    for b in range(T // RB):
        sl = slice(b * RB, (b + 1) * RB)
        prc = pr[sl]
        gt = prr > prc
        tcol = lax.broadcasted_iota(jnp.int32, (RB, T), 1)
        trow = lax.broadcasted_iota(jnp.int32, (RB, T), 0) + b * RB
        eqlt = jnp.logical_and(prr == prc, tcol < trow)
        cmp = jnp.where(jnp.logical_or(gt, eqlt), 1.0, 0.0)
        cnt = jnp.dot(cmp, me, preferred_element_type=jnp.float32)
        e_iota8 = lax.broadcasted_iota(jnp.int32, (RB, E), 1)
        i0b = i0[sl]
        i1b = i1[sl]
        pos0 = jnp.sum(jnp.where(e_iota8 == i0b, cnt, 0.0), axis=1,
                       keepdims=True).astype(jnp.int32)
        pos1 = jnp.sum(jnp.where(e_iota8 == i1b, cnt, 0.0), axis=1,
                       keepdims=True).astype(jnp.int32)
        s0 = jnp.minimum(pos0, CAP - 1)
        s1 = jnp.minimum(pos1, CAP - 1)
        k0 = pos0 < CAP
        k1 = pos1 < CAP
        d0_ref[sl] = jnp.where(k0, i0b * CP + s0, DUMP)
        d1_ref[sl] = jnp.where(k1, i1b * CP + s1, DUMP)
        wb0_ref[sl] = jnp.broadcast_to(jnp.where(k0, w0[sl], 0.0), (RB, 128))
        wb1_ref[sl] = jnp.broadcast_to(jnp.where(k1, w1[sl], 0.0), (RB, 128))


_route_call = pl.pallas_call(
    _route_body,
    out_shape=[
        jax.ShapeDtypeStruct((T, 1), jnp.int32),
        jax.ShapeDtypeStruct((T, 1), jnp.int32),
        jax.ShapeDtypeStruct((T, 128), jnp.float32),
        jax.ShapeDtypeStruct((T, 128), jnp.float32),
        jax.ShapeDtypeStruct((1, E), jnp.float32),
    ],
)


_MESH = plsc.VectorSubcoreMesh(core_axis_name="c", subcore_axis_name="s")
_INFO = plsc.get_sparse_core_info()
_NC = _INFO.num_cores


@functools.partial(
    pl.kernel,
    mesh=_MESH,
    out_type=(
        jax.ShapeDtypeStruct((E * CP, DIM), jnp.float32),
        jax.ShapeDtypeStruct((E * CP, 128), jnp.float32),
    ),
    scratch_types=[
        pltpu.VMEM((TW, DIM), jnp.float32),
        pltpu.VMEM((TW,), jnp.int32),
        pltpu.VMEM((TW,), jnp.int32),
        pltpu.VMEM((TW, 128), jnp.float32),
        pltpu.VMEM((TW, 128), jnp.float32),
        pltpu.SemaphoreType.DMA,
    ],
)
def _scatter_call(x_hbm, d0_hbm, d1_hbm, wb0_hbm, wb1_hbm, ei_hbm, sw_hbm,
                  x_v, d0_v, d1_v, w0_v, w1_v, sem):
    wid = lax.axis_index("s") * _NC + lax.axis_index("c")

    @pl.when(wid < NSUB)
    def _():
        base = wid * TW
        pltpu.sync_copy(x_hbm.at[pl.ds(base, TW)], x_v)
        pltpu.sync_copy(d0_hbm.at[pl.ds(base, TW)], d0_v)
        pltpu.sync_copy(d1_hbm.at[pl.ds(base, TW)], d1_v)
        pltpu.sync_copy(wb0_hbm.at[pl.ds(base, TW)], w0_v)
        pltpu.sync_copy(wb1_hbm.at[pl.ds(base, TW)], w1_v)
        cp0 = pltpu.async_copy(x_v, ei_hbm.at[d0_v], sem)
        cp1 = pltpu.async_copy(x_v, ei_hbm.at[d1_v], sem)
        cp2 = pltpu.async_copy(w0_v, sw_hbm.at[d0_v], sem)
        cp3 = pltpu.async_copy(w1_v, sw_hbm.at[d1_v], sem)
        cp0.wait()
        cp1.wait()
        cp2.wait()
        cp3.wait()


_SQRT1_2 = 0.7071067811865476
_HN = H // HT


def _ffn_body(cnt_ref, x_ref, w1_ref, b1_ref, w2_ref, b2_ref, sw_ref, o_ref):
    h = pl.program_id(1)
    cnt = cnt_ref[0, 0, 0].astype(jnp.int32)
    rows = lax.broadcasted_iota(jnp.int32, (CP, 1), 0)
    live = rows < cnt
    xb = jnp.where(live, x_ref[0], 0.0).astype(jnp.bfloat16)
    hb = jnp.dot(xb, w1_ref[0].astype(jnp.bfloat16),
                 preferred_element_type=jnp.float32) + b1_ref[0]
    hb = 0.5 * hb * (1.0 + lax.erf(hb * _SQRT1_2))
    yb = jnp.dot(hb.astype(jnp.bfloat16), w2_ref[0].astype(jnp.bfloat16),
                 preferred_element_type=jnp.float32)

    if _HN == 1:
        o_ref[0] = (yb + b2_ref[0]) * jnp.where(live, sw_ref[0][:, 0:1], 0.0)
    else:
        @pl.when(h == 0)
        def _():
            o_ref[0] = yb + b2_ref[0]

        @pl.when(jnp.logical_and(h > 0, h < _HN - 1))
        def _():
            o_ref[0] = o_ref[0] + yb

        @pl.when(h == _HN - 1)
        def _():
            o_ref[0] = (o_ref[0] + yb) * sw_ref[0][:, 0:1]


_ffn_call = pl.pallas_call(
    _ffn_body,
    grid=(E, _HN),
    in_specs=[
        pl.BlockSpec((1, 1, 1), lambda e, h: (e, 0, 0)),
        pl.BlockSpec((1, CP, DIM), lambda e, h: (e, 0, 0)),
        pl.BlockSpec((1, DIM, HT), lambda e, h: (e, 0, h)),
        pl.BlockSpec((1, 1, HT), lambda e, h: (e, 0, h)),
        pl.BlockSpec((1, HT, DIM), lambda e, h: (e, h, 0)),
        pl.BlockSpec((1, 1, DIM), lambda e, h: (e, 0, 0)),
        pl.BlockSpec((1, CP, 128), lambda e, h: (e, 0, 0)),
    ],
    out_specs=pl.BlockSpec((1, CP, DIM), lambda e, h: (e, 0, 0)),
    out_shape=jax.ShapeDtypeStruct((E, CP, DIM), jnp.float32),
    compiler_params=pltpu.CompilerParams(
        dimension_semantics=("parallel", "arbitrary")),
)


@functools.partial(
    pl.kernel,
    mesh=_MESH,
    out_type=jax.ShapeDtypeStruct((T, DIM), jnp.float32),
    scratch_types=[
        pltpu.VMEM((TW, DIM), jnp.float32),
        pltpu.VMEM((TW, DIM), jnp.float32),
        pltpu.VMEM((TW,), jnp.int32),
        pltpu.VMEM((TW,), jnp.int32),
        pltpu.SemaphoreType.DMA,
    ],
)
def _combine_call(y_hbm, c0_hbm, c1_hbm, out_hbm, r0, r1, i0_v, i1_v, sem):
    wid = lax.axis_index("s") * _NC + lax.axis_index("c")

    @pl.when(wid < NSUB)
    def _():
        base = wid * TW
        pltpu.sync_copy(c0_hbm.at[pl.ds(base, TW)], i0_v)
        pltpu.sync_copy(c1_hbm.at[pl.ds(base, TW)], i1_v)
        cp0 = pltpu.async_copy(y_hbm.at[i0_v], r0, sem)
        cp1 = pltpu.async_copy(y_hbm.at[i1_v], r1, sem)
        cp0.wait()
        cp1.wait()

        def body(i, carry):
            for c in range(DIM // 16):
                sl = pl.ds(c * 16, 16)
                r0[i, sl] = r0[i, sl] + r1[i, sl]
            return carry

        lax.fori_loop(0, TW, body, 0)
        pltpu.sync_copy(r0, out_hbm.at[pl.ds(base, TW)])


def kernel(x, Wg, bg, W1, b1, W2, b2):
    x_flat = x.reshape(T, DIM)
    d0, d1, wb0, wb1, counts = _route_call(x_flat, Wg, bg.reshape(1, E))
    expert_in, sw = _scatter_call(x_flat, d0.reshape(T), d1.reshape(T),
                                  wb0, wb1)
    y = _ffn_call(counts.reshape(E, 1, 1), expert_in.reshape(E, CP, DIM),
                  W1, b1.reshape(E, 1, H), W2, b2.reshape(E, 1, DIM),
                  sw.reshape(E, CP, 128))
    out = _combine_call(y.reshape(E * CP, DIM), d0.reshape(T), d1.reshape(T))
    return out.reshape(B, N, DIM)


# speedup vs baseline: 2.7607x; 1.0011x over previous
import functools
import math

import jax
import jax.numpy as jnp
from jax import lax
from jax.experimental import pallas as pl
from jax.experimental.pallas import tpu as pltpu
from jax.experimental.pallas import tpu_sc as plsc

DIM = 768
E = 8
K = 2
H = 4 * DIM
B, N = 8, 196
T = B * N
CAP = int(math.ceil(1.25 * T * K / E))
CP = 512
DUMP = CP - 1
NSUB = 28
TW = T // NSUB
RB = 392


def _route_body(x_ref, wg_ref, bg_ref, d0_ref, d1_ref, wb0_ref, wb1_ref,
                cnt_ref):
    logits = jnp.dot(x_ref[...], wg_ref[...], preferred_element_type=jnp.float32)
    logits = logits + bg_ref[...]
    gmax = jnp.max(logits, axis=1, keepdims=True)
    z = jnp.exp(logits - gmax)
    gates = z / jnp.sum(z, axis=1, keepdims=True)
    e_iota = lax.broadcasted_iota(jnp.int32, (T, E), 1)
    v0 = jnp.max(gates, axis=1, keepdims=True)
    i0 = jnp.min(jnp.where(gates == v0, e_iota, E), axis=1, keepdims=True)
    g1 = jnp.where(e_iota == i0, -1.0, gates)
    v1 = jnp.max(g1, axis=1, keepdims=True)
    i1 = jnp.min(jnp.where(g1 == v1, e_iota, E), axis=1, keepdims=True)
    ws = v0 + v1
    w0 = v0 / ws
    w1 = v1 / ws
    me = jnp.logical_or(e_iota == i0, e_iota == i1).astype(jnp.float32)
    cnt_ref[...] = jnp.minimum(jnp.sum(me, axis=0, keepdims=True), float(CAP))
    pr = v0
    prr_parts = []
    for b in range(T // RB):
        crow = lax.broadcasted_iota(jnp.int32, (T, RB), 0)
        ccol = lax.broadcasted_iota(jnp.int32, (T, RB), 1) + b * RB
        ident = (crow == ccol).astype(jnp.float32)
        prr_parts.append(lax.dot_general(
            pr, ident, (((0,), (0,)), ((), ())),
            precision=lax.Precision.HIGHEST,
            preferred_element_type=jnp.float32))
    prr = jnp.concatenate(prr_parts, axis=1)
    for b in range(T // RB):
        sl = slice(b * RB, (b + 1) * RB)
        prc = pr[sl]
        gt = prr > prc
        tcol = lax.broadcasted_iota(jnp.int32, (RB, T), 1)
        trow = lax.broadcasted_iota(jnp.int32, (RB, T), 0) + b * RB
        eqlt = jnp.logical_and(prr == prc, tcol < trow)
        cmp = jnp.where(jnp.logical_or(gt, eqlt), 1.0, 0.0)
        cnt = jnp.dot(cmp, me, preferred_element_type=jnp.float32)
        e_iota8 = lax.broadcasted_iota(jnp.int32, (RB, E), 1)
        i0b = i0[sl]
        i1b = i1[sl]
        pos0 = jnp.sum(jnp.where(e_iota8 == i0b, cnt, 0.0), axis=1,
                       keepdims=True).astype(jnp.int32)
        pos1 = jnp.sum(jnp.where(e_iota8 == i1b, cnt, 0.0), axis=1,
                       keepdims=True).astype(jnp.int32)
        s0 = jnp.minimum(pos0, CAP - 1)
        s1 = jnp.minimum(pos1, CAP - 1)
        k0 = pos0 < CAP
        k1 = pos1 < CAP
        d0_ref[sl] = jnp.where(k0, i0b * CP + s0, DUMP)
        d1_ref[sl] = jnp.where(k1, i1b * CP + s1, DUMP)
        wb0_ref[sl] = jnp.broadcast_to(jnp.where(k0, w0[sl], 0.0), (RB, 128))
        wb1_ref[sl] = jnp.broadcast_to(jnp.where(k1, w1[sl], 0.0), (RB, 128))


_route_call = pl.pallas_call(
    _route_body,
    out_shape=[
        jax.ShapeDtypeStruct((T, 1), jnp.int32),
        jax.ShapeDtypeStruct((T, 1), jnp.int32),
        jax.ShapeDtypeStruct((T, 128), jnp.float32),
        jax.ShapeDtypeStruct((T, 128), jnp.float32),
        jax.ShapeDtypeStruct((1, E), jnp.float32),
    ],
)


_MESH = plsc.VectorSubcoreMesh(core_axis_name="c", subcore_axis_name="s")
_INFO = plsc.get_sparse_core_info()
_NC = _INFO.num_cores


@functools.partial(
    pl.kernel,
    mesh=_MESH,
    out_type=(
        jax.ShapeDtypeStruct((E * CP, DIM), jnp.float32),
        jax.ShapeDtypeStruct((E * CP, 128), jnp.float32),
    ),
    scratch_types=[
        pltpu.VMEM((TW, DIM), jnp.float32),
        pltpu.VMEM((TW,), jnp.int32),
        pltpu.VMEM((TW,), jnp.int32),
        pltpu.VMEM((TW, 128), jnp.float32),
        pltpu.VMEM((TW, 128), jnp.float32),
        pltpu.SemaphoreType.DMA,
    ],
)
def _scatter_call(x_hbm, d0_hbm, d1_hbm, wb0_hbm, wb1_hbm, ei_hbm, sw_hbm,
                  x_v, d0_v, d1_v, w0_v, w1_v, sem):
    wid = lax.axis_index("s") * _NC + lax.axis_index("c")

    @pl.when(wid < NSUB)
    def _():
        base = wid * TW
        pltpu.sync_copy(x_hbm.at[pl.ds(base, TW)], x_v)
        pltpu.sync_copy(d0_hbm.at[pl.ds(base, TW)], d0_v)
        pltpu.sync_copy(d1_hbm.at[pl.ds(base, TW)], d1_v)
        pltpu.sync_copy(wb0_hbm.at[pl.ds(base, TW)], w0_v)
        pltpu.sync_copy(wb1_hbm.at[pl.ds(base, TW)], w1_v)
        cp0 = pltpu.async_copy(x_v, ei_hbm.at[d0_v], sem)
        cp1 = pltpu.async_copy(x_v, ei_hbm.at[d1_v], sem)
        cp2 = pltpu.async_copy(w0_v, sw_hbm.at[d0_v], sem)
        cp3 = pltpu.async_copy(w1_v, sw_hbm.at[d1_v], sem)
        cp0.wait()
        cp1.wait()
        cp2.wait()
        cp3.wait()


_SQRT1_2 = 0.7071067811865476


def _ffn_body(cnt_ref, x_ref, w1_ref, b1_ref, w2_ref, b2_ref, sw_ref, o_ref):
    cnt = cnt_ref[0, 0, 0].astype(jnp.int32)
    rows = lax.broadcasted_iota(jnp.int32, (CP, 1), 0)
    live = rows < cnt
    xb = jnp.where(live, x_ref[0], 0.0).astype(jnp.bfloat16)
    hb = jnp.dot(xb, w1_ref[0].astype(jnp.bfloat16),
                 preferred_element_type=jnp.float32) + b1_ref[0]
    hb = 0.5 * hb * (1.0 + lax.erf(hb * _SQRT1_2))
    yb = jnp.dot(hb.astype(jnp.bfloat16), w2_ref[0].astype(jnp.bfloat16),
                 preferred_element_type=jnp.float32)
    o_ref[0] = (yb + b2_ref[0]) * jnp.where(live, sw_ref[0][:, 0:1], 0.0)


_ffn_call = pl.pallas_call(
    _ffn_body,
    grid=(E,),
    in_specs=[
        pl.BlockSpec((1, 1, 1), lambda e: (e, 0, 0)),
        pl.BlockSpec((1, CP, DIM), lambda e: (e, 0, 0)),
        pl.BlockSpec((1, DIM, H), lambda e: (e, 0, 0)),
        pl.BlockSpec((1, 1, H), lambda e: (e, 0, 0)),
        pl.BlockSpec((1, H, DIM), lambda e: (e, 0, 0)),
        pl.BlockSpec((1, 1, DIM), lambda e: (e, 0, 0)),
        pl.BlockSpec((1, CP, 128), lambda e: (e, 0, 0)),
    ],
    out_specs=pl.BlockSpec((1, CP, DIM), lambda e: (e, 0, 0)),
    out_shape=jax.ShapeDtypeStruct((E, CP, DIM), jnp.float32),
    compiler_params=pltpu.CompilerParams(
        dimension_semantics=("arbitrary",)),
)


@functools.partial(
    pl.kernel,
    mesh=_MESH,
    out_type=jax.ShapeDtypeStruct((T, DIM), jnp.float32),
    scratch_types=[
        pltpu.VMEM((TW, DIM), jnp.float32),
        pltpu.VMEM((TW, DIM), jnp.float32),
        pltpu.VMEM((TW,), jnp.int32),
        pltpu.VMEM((TW,), jnp.int32),
        pltpu.SemaphoreType.DMA,
    ],
)
def _combine_call(y_hbm, c0_hbm, c1_hbm, out_hbm, r0, r1, i0_v, i1_v, sem):
    wid = lax.axis_index("s") * _NC + lax.axis_index("c")

    @pl.when(wid < NSUB)
    def _():
        base = wid * TW
        pltpu.sync_copy(c0_hbm.at[pl.ds(base, TW)], i0_v)
        pltpu.sync_copy(c1_hbm.at[pl.ds(base, TW)], i1_v)
        cp0 = pltpu.async_copy(y_hbm.at[i0_v], r0, sem)
        cp1 = pltpu.async_copy(y_hbm.at[i1_v], r1, sem)
        cp0.wait()
        cp1.wait()

        def body(i, carry):
            for c in range(DIM // 16):
                sl = pl.ds(c * 16, 16)
                r0[i, sl] = r0[i, sl] + r1[i, sl]
            return carry

        lax.fori_loop(0, TW, body, 0)
        pltpu.sync_copy(r0, out_hbm.at[pl.ds(base, TW)])


def kernel(x, Wg, bg, W1, b1, W2, b2):
    x_flat = x.reshape(T, DIM)
    d0, d1, wb0, wb1, counts = _route_call(x_flat, Wg, bg.reshape(1, E))
    expert_in, sw = _scatter_call(x_flat, d0.reshape(T), d1.reshape(T),
                                  wb0, wb1)
    y = _ffn_call(counts.reshape(E, 1, 1), expert_in.reshape(E, CP, DIM),
                  W1, b1.reshape(E, 1, H), W2, b2.reshape(E, 1, DIM),
                  sw.reshape(E, CP, 128))
    out = _combine_call(y.reshape(E * CP, DIM), d0.reshape(T), d1.reshape(T))
    return out.reshape(B, N, DIM)
